# baseline (speedup 1.0000x reference)
import functools
import math

import jax
import jax.numpy as jnp
from jax import lax
from jax.experimental import pallas as pl
from jax.experimental.pallas import tpu as pltpu
from jax.experimental.pallas import tpu_sc as plsc

D_MODEL = 64
SCALE = math.sqrt(D_MODEL)

NC = 2
NS = 16
NW = NC * NS
IDXW = 128
SUB = 2
CHUNK = IDXW * SUB


@jax.jit
def _embed(xp2, xh2, tp):
    B = xp2.shape[0] * IDXW
    b_per_w = B // NW
    n_chunks = b_per_w // CHUNK
    idx_rows_per_w = b_per_w // IDXW

    mesh = plsc.VectorSubcoreMesh(core_axis_name="c", subcore_axis_name="s")

    @functools.partial(
        pl.kernel,
        mesh=mesh,
        out_type=jax.ShapeDtypeStruct((B, D_MODEL), jnp.float32),
        scratch_types=[
            pltpu.VMEM((idx_rows_per_w, IDXW), jnp.int32),
            pltpu.VMEM((idx_rows_per_w, IDXW), jnp.int32),
            pltpu.VMEM((CHUNK, 128), jnp.float32),
            pltpu.VMEM((CHUNK, D_MODEL), jnp.float32),
            pltpu.SemaphoreType.DMA,
        ],
        compiler_params=pltpu.CompilerParams(
            use_tc_tiling_on_sc=True, needs_layout_passes=False
        ),
    )
    def body(xp_hbm, xh_hbm, tp_hbm, out_hbm, pv, hv, rowsg, out64, sem):
        wid = lax.axis_index("s") * NC + lax.axis_index("c")
        row_base = wid * b_per_w
        idx_base = wid * idx_rows_per_w

        pltpu.sync_copy(xp_hbm.at[pl.ds(idx_base, idx_rows_per_w)], pv)
        pltpu.sync_copy(xh_hbm.at[pl.ds(idx_base, idx_rows_per_w)], hv)

        lanes = lax.iota(jnp.int32, 16)

        def chunk_body(g, carry):
            off = row_base + g * CHUNK
            copies = [
                pltpu.async_copy(
                    tp_hbm.at[pv.at[g * SUB + j]],
                    rowsg.at[pl.ds(j * IDXW, IDXW)],
                    sem,
                )
                for j in range(SUB)
            ]
            for c in copies:
                c.wait()

            def group_body(q, c2):
                h16 = hv[g * SUB + q // 8, pl.ds((q % 8) * 16, 16)]
                rowv = q * 16 + lanes
                colbase = h16 * D_MODEL
                for col in range(D_MODEL):
                    vals = plsc.load_gather(rowsg, [rowv, colbase + col])
                    plsc.store_scatter(
                        out64,
                        [rowv, jnp.full((16,), col, jnp.int32)],
                        vals * SCALE,
                    )
                return c2

            lax.fori_loop(0, CHUNK // 16, group_body, 0)
            pltpu.sync_copy(out64, out_hbm.at[pl.ds(off, CHUNK)])
            return carry

        lax.fori_loop(0, n_chunks, chunk_body, 0)

    return body(xp2, xh2, tp)


def kernel(x, table):
    xf = x.reshape(-1).astype(jnp.int32)
    xp2 = (xf >> 1).reshape(-1, IDXW)
    xh2 = (xf & 1).reshape(-1, IDXW)
    tp = table.reshape(table.shape[0] // 2, 2 * table.shape[1])
    out = _embed(xp2, xh2, tp)
    return out.reshape(x.shape + (table.shape[1],))


# speedup vs baseline: 1.8412x; 1.8412x over previous
import functools
import math

import jax
import jax.numpy as jnp
from jax import lax
from jax.experimental import pallas as pl
from jax.experimental.pallas import tpu as pltpu
from jax.experimental.pallas import tpu_sc as plsc

D_MODEL = 64
SCALE = math.sqrt(D_MODEL)

NC = 2
NS = 16
NW = NC * NS
IDXW = 128
SUB = 2
CHUNK = IDXW * SUB


@jax.jit
def _embed(xi2, t128):
    B = xi2.shape[0] * IDXW
    b_per_w = B // NW
    n_chunks = b_per_w // CHUNK
    idx_rows_per_w = b_per_w // IDXW

    mesh = plsc.VectorSubcoreMesh(core_axis_name="c", subcore_axis_name="s")

    @functools.partial(
        pl.kernel,
        mesh=mesh,
        out_type=jax.ShapeDtypeStruct((B, D_MODEL), jnp.float32),
        scratch_types=[
            pltpu.VMEM((idx_rows_per_w, IDXW), jnp.int32),
            pltpu.VMEM((CHUNK, 128), jnp.float32),
            pltpu.VMEM((CHUNK, D_MODEL), jnp.float32),
            pltpu.SemaphoreType.DMA,
        ],
        compiler_params=pltpu.CompilerParams(
            use_tc_tiling_on_sc=True, needs_layout_passes=False
        ),
    )
    def body(xi_hbm, t_hbm, out_hbm, iv, rowsg, out64, sem):
        wid = lax.axis_index("s") * NC + lax.axis_index("c")
        row_base = wid * b_per_w
        idx_base = wid * idx_rows_per_w

        pltpu.sync_copy(xi_hbm.at[pl.ds(idx_base, idx_rows_per_w)], iv)

        def chunk_body(g, carry):
            off = row_base + g * CHUNK
            copies = [
                pltpu.async_copy(
                    t_hbm.at[iv.at[g * SUB + j]],
                    rowsg.at[pl.ds(j * IDXW, IDXW)],
                    sem,
                )
                for j in range(SUB)
            ]
            for c in copies:
                c.wait()

            def scale_body(r, c2):
                for k in range(D_MODEL // 16):
                    sl = pl.ds(k * 16, 16)
                    out64[r, sl] = rowsg[r, sl] * SCALE
                return c2

            lax.fori_loop(0, CHUNK, scale_body, 0, unroll=4)
            pltpu.sync_copy(out64, out_hbm.at[pl.ds(off, CHUNK)])
            return carry

        lax.fori_loop(0, n_chunks, chunk_body, 0)

    return body(xi2, t128)


def kernel(x, table):
    xf = x.reshape(-1).astype(jnp.int32)
    xi2 = xf.reshape(-1, IDXW)
    t128 = jnp.concatenate([table, table], axis=1)
    out = _embed(xi2, t128)
    return out.reshape(x.shape + (table.shape[1],))


# speedup vs baseline: 2.1862x; 1.1873x over previous
import functools
import math

import jax
import jax.numpy as jnp
from jax import lax
from jax.experimental import pallas as pl
from jax.experimental.pallas import tpu as pltpu
from jax.experimental.pallas import tpu_sc as plsc

D_MODEL = 64
SCALE = math.sqrt(D_MODEL)

NC = 2
NS = 16
NW = NC * NS
BLK = 200
GROUPS = tuple(list(range(0, BLK - 16, 16)) + [BLK - 16])


def _bcast(vec, lane):
    dn = lax.GatherDimensionNumbers(
        offset_dims=(), collapsed_slice_dims=(0,), start_index_map=(0,)
    )
    idx = jnp.full((16, 1), lane, jnp.int32)
    return lax.gather(
        vec, idx, dn, slice_sizes=(1,),
        mode=lax.GatherScatterMode.PROMISE_IN_BOUNDS,
    )


@functools.partial(jax.jit, static_argnums=(2, 3))
def _embed(xf, tp, a_dim, b_dim):
    B = xf.shape[0]
    b_per_w = B // NW
    blocks_per_w = b_per_w // BLK

    mesh = plsc.VectorSubcoreMesh(core_axis_name="c", subcore_axis_name="s")

    @functools.partial(
        pl.kernel,
        mesh=mesh,
        out_type=jax.ShapeDtypeStruct((a_dim, b_dim, D_MODEL), jnp.float32),
        scratch_types=[
            pltpu.VMEM((b_per_w,), jnp.int32),
            pltpu.VMEM((256,), jnp.int32),
            pltpu.VMEM((BLK, 128), jnp.float32),
            pltpu.VMEM((BLK, D_MODEL), jnp.float32),
            pltpu.SemaphoreType.DMA,
        ],
        compiler_params=pltpu.CompilerParams(
            use_tc_tiling_on_sc=True, needs_layout_passes=False
        ),
    )
    def body(xf_hbm, tp_hbm, out_hbm, iv, pv, rowsg, out64, sem):
        wid = lax.axis_index("s") * NC + lax.axis_index("c")
        row_base = wid * b_per_w
        blk_base = wid * blocks_per_w

        pltpu.sync_copy(xf_hbm.at[pl.ds(row_base, b_per_w)], iv)

        def block_body(a, carry):
            base = a * BLK

            def pv_body(g, c2):
                gs = g * 16
                pv[pl.ds(gs, 16)] = iv[pl.ds(base + gs, 16)] >> 1
                return c2

            lax.fori_loop(0, BLK // 16, pv_body, 0)
            pv[pl.ds(BLK - 16, 16)] = iv[pl.ds(base + BLK - 16, 16)] >> 1

            copies = [
                pltpu.async_copy(
                    tp_hbm.at[pv.at[pl.ds(0, 128)]],
                    rowsg.at[pl.ds(0, 128)],
                    sem,
                ),
                pltpu.async_copy(
                    tp_hbm.at[pv.at[pl.ds(128, BLK - 128)]],
                    rowsg.at[pl.ds(128, BLK - 128)],
                    sem,
                ),
            ]
            for c in copies:
                c.wait()

            def group_body(g, c2):
                hsel(g * 16)
                return c2

            def hsel(gs):
                h16 = iv[pl.ds(base + gs, 16)] & 1
                for j in range(16):
                    hb = _bcast(h16, j)
                    r = gs + j
                    for k in range(D_MODEL // 16):
                        alo = rowsg[r, pl.ds(k * 16, 16)]
                        ahi = rowsg[r, pl.ds(64 + k * 16, 16)]
                        out64[r, pl.ds(k * 16, 16)] = (
                            jnp.where(hb > 0, ahi, alo) * SCALE
                        )

            lax.fori_loop(0, BLK // 16, group_body, 0)
            hsel(BLK - 16)

            pltpu.sync_copy(out64, out_hbm.at[blk_base + a])
            return carry

        lax.fori_loop(0, blocks_per_w, block_body, 0)

    return body(xf, tp)


def kernel(x, table):
    xf = x.reshape(-1).astype(jnp.int32)
    tp = table.reshape(table.shape[0] // 2, 2 * table.shape[1])
    return _embed(xf, tp, x.shape[0], x.shape[1])
